# baseline (speedup 1.0000x reference)
import functools

import jax
import jax.numpy as jnp
from jax import lax
from jax.experimental import pallas as pl
from jax.experimental.pallas import tpu as pltpu
from jax.experimental.pallas import tpu_sc as plsc

N = 10000
D = 128
E = 160000

NC = 2
NS = 16
NW = NC * NS
N_PAD = 10240
E_PER_W = 5120
E_PAD = NW * E_PER_W
G = 128
NBATCH = E_PER_W // G
ROWS_PER_SUB = N_PAD // NS
DT = 0.1


def _sc_agg_body(uprime, src, dst, zout, sidx, didx, stag0, stag1, zsh,
                 isem, gsa, gsb, ssa, ssb):
    c = lax.axis_index("c")
    s = lax.axis_index("s")
    wid = c * NS + s
    ebase = wid * E_PER_W

    idx_descs = [
        pltpu.async_copy(src.at[pl.ds(ebase, E_PER_W)], sidx, isem),
        pltpu.async_copy(dst.at[pl.ds(ebase, E_PER_W)], didx, isem),
    ]

    def zero_row(i, _):
        for k in range(D // 16):
            stag0[i, pl.ds(k * 16, 16)] = jnp.zeros((16,), jnp.float32)
        return 0

    lax.fori_loop(0, G, zero_row, 0)
    zdescs = [
        pltpu.async_copy(stag0, zsh.at[pl.ds(s * ROWS_PER_SUB + j * G, G)], gsa)
        for j in range(ROWS_PER_SUB // G)
    ]
    for d in zdescs:
        d.wait()
    for d in idx_descs:
        d.wait()
    plsc.subcore_barrier()

    bufs = (stag0, stag1)
    gsem = (gsa, gsb)
    ssem = (ssa, ssb)
    gd = [None] * NBATCH
    sd = [None] * NBATCH
    gd[0] = pltpu.async_copy(uprime.at[sidx.at[pl.ds(0, G)]], bufs[0], gsem[0])
    for i in range(NBATCH):
        if i + 1 < NBATCH:
            if i >= 1:
                sd[i - 1].wait()
            b = (i + 1) % 2
            gd[i + 1] = pltpu.async_copy(
                uprime.at[sidx.at[pl.ds((i + 1) * G, G)]], bufs[b], gsem[b])
        gd[i].wait()
        b = i % 2
        sd[i] = pltpu.async_copy(
            bufs[b], zsh.at[didx.at[pl.ds(i * G, G)]], ssem[b], add=True)
    sd[NBATCH - 2].wait()
    sd[NBATCH - 1].wait()
    plsc.subcore_barrier()

    r0 = s * ROWS_PER_SUB
    pltpu.sync_copy(zsh.at[pl.ds(r0, ROWS_PER_SUB)], zout.at[c, pl.ds(r0, ROWS_PER_SUB)])


@functools.cache
def _get_sc_agg():
    return pl.kernel(
        _sc_agg_body,
        out_type=jax.ShapeDtypeStruct((NC, N_PAD, D), jnp.float32),
        mesh=plsc.VectorSubcoreMesh(
            core_axis_name="c", subcore_axis_name="s", num_cores=NC, num_subcores=NS
        ),
        scratch_types=[
            pltpu.VMEM((E_PER_W,), jnp.int32),
            pltpu.VMEM((E_PER_W,), jnp.int32),
            pltpu.VMEM((G, D), jnp.float32),
            pltpu.VMEM((G, D), jnp.float32),
            pltpu.VMEM_SHARED((N_PAD, D), jnp.float32),
            pltpu.SemaphoreType.DMA,
            pltpu.SemaphoreType.DMA,
            pltpu.SemaphoreType.DMA,
            pltpu.SemaphoreType.DMA,
            pltpu.SemaphoreType.DMA,
        ],
        name="gcn_edge_agg",
    )


BR = 640


def _psi(z_ref, up_ref, dinv_ref, w_ref, b_ref, tf_ref):
    g = dinv_ref[...] * (z_ref[0] + z_ref[1] + up_ref[...])
    h = jnp.dot(g, w_ref[...], preferred_element_type=jnp.float32) + b_ref[...]
    return jnp.maximum(h, 0.0) * tf_ref[...]


def _s1_body(z, yp, y, dinv, w, b, tf, k1_o, u2p_o, acc_o):
    k1 = _psi(z, yp, dinv, w, b, tf)
    k1_o[...] = k1
    u2p_o[...] = dinv[...] * (y[...] + (DT / 3.0) * k1)
    acc_o[...] = y[...] + (DT / 8.0) * k1


def _s2_body(z, u2p, y, k1, acc, dinv, w, b, tf, u3p_o, m_o, acc_o):
    k2 = _psi(z, u2p, dinv, w, b, tf)
    u3p_o[...] = dinv[...] * (y[...] + DT * k2 - (DT / 3.0) * k1[...])
    m_o[...] = k1[...] - k2
    acc_o[...] = acc[...] + (3.0 * DT / 8.0) * k2


def _s3_body(z, u3p, y, m, acc, dinv, w, b, tf, u4p_o, acc_o):
    k3 = _psi(z, u3p, dinv, w, b, tf)
    u4p_o[...] = dinv[...] * (y[...] + DT * (m[...] + k3))
    acc_o[...] = acc[...] + (3.0 * DT / 8.0) * k3


def _s4_body(z, u4p, acc, dinv, w, b, tf, y_o, yp_o):
    k4 = _psi(z, u4p, dinv, w, b, tf)
    y = acc[...] + (DT / 8.0) * k4
    y_o[...] = y
    yp_o[...] = dinv[...] * y


_row = pl.BlockSpec((BR, D), lambda i: (i, 0))
_zs = pl.BlockSpec((NC, BR, D), lambda i: (0, i, 0))
_dv = pl.BlockSpec((BR, 1), lambda i: (i, 0))
_wf = pl.BlockSpec((D, D), lambda i: (0, 0))
_vec = pl.BlockSpec((1, D), lambda i: (0, 0))
_grid = N_PAD // BR
_out = jax.ShapeDtypeStruct((N_PAD, D), jnp.float32)


def _stage_call(body, n_in, n_out):
    return pl.pallas_call(
        body,
        grid=(_grid,),
        in_specs=[_zs] + [_row] * n_in + [_dv, _wf, _vec, _vec],
        out_specs=[_row] * n_out,
        out_shape=[_out] * n_out,
    )


_tc_s1 = _stage_call(_s1_body, 2, 3)
_tc_s2 = _stage_call(_s2_body, 4, 3)
_tc_s3 = _stage_call(_s3_body, 4, 2)
_tc_s4 = _stage_call(_s4_body, 2, 2)


def _integrate(x_pad, src, dst, w, b, tw):
    sc_agg = _get_sc_agg()
    deg = jnp.zeros((N_PAD,), jnp.float32).at[dst].add(1.0) + 1.0
    pad_w = E_PER_W - E // NW
    pad_idx = (N + (jnp.arange(NW * pad_w, dtype=jnp.int32) % (N_PAD - N))
               ).reshape(NW, pad_w)
    src = jnp.concatenate([src.reshape(NW, E // NW), pad_idx], axis=1).reshape(-1)
    dst = jnp.concatenate([dst.reshape(NW, E // NW), pad_idx], axis=1).reshape(-1)
    dinv = lax.rsqrt(deg)[:, None]
    b2 = b[None, :]

    y = x_pad
    yp = dinv * y
    for i in range(10):
        t0 = i * DT
        tfs = [jax.nn.sigmoid((t0 + c * DT) * tw)[None, :]
               for c in (0.0, 1.0 / 3.0, 2.0 / 3.0, 1.0)]
        z = sc_agg(yp, src, dst)
        k1, u2p, acc = _tc_s1(z, yp, y, dinv, w, b2, tfs[0])
        z = sc_agg(u2p, src, dst)
        u3p, m, acc = _tc_s2(z, u2p, y, k1, acc, dinv, w, b2, tfs[1])
        z = sc_agg(u3p, src, dst)
        u4p, acc = _tc_s3(z, u3p, y, m, acc, dinv, w, b2, tfs[2])
        z = sc_agg(u4p, src, dst)
        y, yp = _tc_s4(z, u4p, acc, dinv, w, b2, tfs[3])
    return y[:N]


def kernel(x, pos_edge_index, neg_edge_index, W_pos, b_pos, tw_pos, W_neg, b_neg, tw_neg):
    x_pad = jnp.pad(x, ((0, N_PAD - N), (0, 0)))
    z_pos = _integrate(x_pad, pos_edge_index[0], pos_edge_index[1], W_pos, b_pos, tw_pos)
    z_neg = _integrate(x_pad, neg_edge_index[0], neg_edge_index[1], W_neg, b_neg, tw_neg)
    return (z_pos, z_neg)


# speedup vs baseline: 1.0468x; 1.0468x over previous
import functools

import jax
import jax.numpy as jnp
from jax import lax
from jax.experimental import pallas as pl
from jax.experimental.pallas import tpu as pltpu
from jax.experimental.pallas import tpu_sc as plsc

N = 10000
D = 128
E = 160000

NC = 2
NS = 16
NW = NC * NS
N_PAD = 10240
E_PER_W = 5120
E_PAD = NW * E_PER_W
G = 128
NBATCH = E_PER_W // G
ROWS_PER_SUB = N_PAD // NS
DT = 0.1


def _sc_agg_body(uprime, src, dst, zout, sidx, didx, stag0, stag1, zsh,
                 isem, gsa, gsb, ssa, ssb):
    c = lax.axis_index("c")
    s = lax.axis_index("s")
    wid = c * NS + s
    ebase = wid * E_PER_W

    idx_descs = [
        pltpu.async_copy(src.at[pl.ds(ebase, E_PER_W)], sidx, isem),
        pltpu.async_copy(dst.at[pl.ds(ebase, E_PER_W)], didx, isem),
    ]

    def zero_row(i, _):
        for k in range(D // 16):
            stag0[i, pl.ds(k * 16, 16)] = jnp.zeros((16,), jnp.float32)
        return 0

    lax.fori_loop(0, G, zero_row, 0)
    zdescs = [
        pltpu.async_copy(stag0, zsh.at[pl.ds(s * ROWS_PER_SUB + j * G, G)], gsa)
        for j in range(ROWS_PER_SUB // G)
    ]
    for d in zdescs:
        d.wait()
    for d in idx_descs:
        d.wait()
    plsc.subcore_barrier()

    bufs = (stag0, stag1)
    gsem = (gsa, gsb)
    ssem = (ssa, ssb)
    gd = [None] * NBATCH
    sd = [None] * NBATCH
    gd[0] = pltpu.async_copy(uprime.at[sidx.at[pl.ds(0, G)]], bufs[0], gsem[0])
    for i in range(NBATCH):
        if i + 1 < NBATCH:
            if i >= 1:
                sd[i - 1].wait()
            b = (i + 1) % 2
            gd[i + 1] = pltpu.async_copy(
                uprime.at[sidx.at[pl.ds((i + 1) * G, G)]], bufs[b], gsem[b])
        gd[i].wait()
        b = i % 2
        sd[i] = pltpu.async_copy(
            bufs[b], zsh.at[didx.at[pl.ds(i * G, G)]], ssem[b], add=True)
    sd[NBATCH - 2].wait()
    sd[NBATCH - 1].wait()
    plsc.subcore_barrier()

    r0 = s * ROWS_PER_SUB
    pltpu.sync_copy(zsh.at[pl.ds(r0, ROWS_PER_SUB)], zout.at[c, pl.ds(r0, ROWS_PER_SUB)])


@functools.cache
def _get_sc_agg():
    return pl.kernel(
        _sc_agg_body,
        out_type=jax.ShapeDtypeStruct((NC, N_PAD, D), jnp.float32),
        mesh=plsc.VectorSubcoreMesh(
            core_axis_name="c", subcore_axis_name="s", num_cores=NC, num_subcores=NS
        ),
        scratch_types=[
            pltpu.VMEM((E_PER_W,), jnp.int32),
            pltpu.VMEM((E_PER_W,), jnp.int32),
            pltpu.VMEM((G, D), jnp.float32),
            pltpu.VMEM((G, D), jnp.float32),
            pltpu.VMEM_SHARED((N_PAD, D), jnp.float32),
            pltpu.SemaphoreType.DMA,
            pltpu.SemaphoreType.DMA,
            pltpu.SemaphoreType.DMA,
            pltpu.SemaphoreType.DMA,
            pltpu.SemaphoreType.DMA,
        ],
        name="gcn_edge_agg",
    )


BR = 640


def _psi(z_ref, up_ref, dinv_ref, w_ref, b_ref, tf_ref):
    g = dinv_ref[...] * (z_ref[0] + z_ref[1] + up_ref[...])
    h = jnp.dot(g, w_ref[...], preferred_element_type=jnp.float32) + b_ref[...]
    return jnp.maximum(h, 0.0) * tf_ref[...]


def _s1_body(z, yp, y, dinv, w, b, tf, k1_o, u2p_o, acc_o):
    k1 = _psi(z, yp, dinv, w, b, tf)
    k1_o[...] = k1
    u2p_o[...] = dinv[...] * (y[...] + (DT / 3.0) * k1)
    acc_o[...] = y[...] + (DT / 8.0) * k1


def _s2_body(z, u2p, y, k1, acc, dinv, w, b, tf, u3p_o, m_o, acc_o):
    k2 = _psi(z, u2p, dinv, w, b, tf)
    u3p_o[...] = dinv[...] * (y[...] + DT * k2 - (DT / 3.0) * k1[...])
    m_o[...] = k1[...] - k2
    acc_o[...] = acc[...] + (3.0 * DT / 8.0) * k2


def _s3_body(z, u3p, y, m, acc, dinv, w, b, tf, u4p_o, acc_o):
    k3 = _psi(z, u3p, dinv, w, b, tf)
    u4p_o[...] = dinv[...] * (y[...] + DT * (m[...] + k3))
    acc_o[...] = acc[...] + (3.0 * DT / 8.0) * k3


def _s4_body(z, u4p, acc, dinv, w, b, tf, y_o, yp_o):
    k4 = _psi(z, u4p, dinv, w, b, tf)
    y = acc[...] + (DT / 8.0) * k4
    y_o[...] = y
    yp_o[...] = dinv[...] * y


_row = pl.BlockSpec((BR, D), lambda i: (i, 0))
_zs = pl.BlockSpec((NC, BR, D), lambda i: (0, i, 0))
_dv = pl.BlockSpec((BR, 1), lambda i: (i, 0))
_wf = pl.BlockSpec((D, D), lambda i: (0, 0))
_vec = pl.BlockSpec((1, D), lambda i: (0, 0))
_grid = N_PAD // BR
_out = jax.ShapeDtypeStruct((N_PAD, D), jnp.float32)


def _stage_call(body, n_in, n_out):
    return pl.pallas_call(
        body,
        grid=(_grid,),
        in_specs=[_zs] + [_row] * n_in + [_dv, _wf, _vec, _vec],
        out_specs=[_row] * n_out,
        out_shape=[_out] * n_out,
    )


_tc_s1 = _stage_call(_s1_body, 2, 3)
_tc_s2 = _stage_call(_s2_body, 4, 3)
_tc_s3 = _stage_call(_s3_body, 4, 2)
_tc_s4 = _stage_call(_s4_body, 2, 2)


def _pad_edges(src, dst):
    pad_w = E_PER_W - E // NW
    pad_idx = (N + (jnp.arange(NW * pad_w, dtype=jnp.int32) % (N_PAD - N))
               ).reshape(NW, pad_w)
    src = jnp.concatenate([src.reshape(NW, E // NW), pad_idx], axis=1).reshape(-1)
    dst = jnp.concatenate([dst.reshape(NW, E // NW), pad_idx], axis=1).reshape(-1)
    return src, dst


def kernel(x, pos_edge_index, neg_edge_index, W_pos, b_pos, tw_pos, W_neg, b_neg, tw_neg):
    sc_agg = _get_sc_agg()
    x_pad = jnp.pad(x, ((0, N_PAD - N), (0, 0)))
    ones = jnp.ones((N_PAD, D), jnp.float32)

    edges, dinvs, b2s, tws, ws = [], [], [], [], []
    for src, dst, w, b, tw in (
        (pos_edge_index[0], pos_edge_index[1], W_pos, b_pos, tw_pos),
        (neg_edge_index[0], neg_edge_index[1], W_neg, b_neg, tw_neg),
    ):
        src, dst = _pad_edges(src, dst)
        zd = sc_agg(ones, src, dst)
        deg = zd[0, :, 0] + zd[1, :, 0] + 1.0
        edges.append((src, dst))
        dinvs.append(lax.rsqrt(deg)[:, None])
        b2s.append(b[None, :])
        tws.append(tw)
        ws.append(w)

    y = [x_pad, x_pad]
    yp = [dinvs[q] * x_pad for q in range(2)]
    k1 = [None, None]
    u2p = [None, None]
    u3p = [None, None]
    u4p = [None, None]
    m = [None, None]
    acc = [None, None]
    z = [None, None]
    for i in range(10):
        t0 = i * DT
        tfs = [[jax.nn.sigmoid((t0 + c * DT) * tws[q])[None, :]
                for c in (0.0, 1.0 / 3.0, 2.0 / 3.0, 1.0)] for q in range(2)]
        for q in range(2):
            z[q] = sc_agg(yp[q], *edges[q])
        for q in range(2):
            k1[q], u2p[q], acc[q] = _tc_s1(z[q], yp[q], y[q], dinvs[q], ws[q], b2s[q], tfs[q][0])
        for q in range(2):
            z[q] = sc_agg(u2p[q], *edges[q])
        for q in range(2):
            u3p[q], m[q], acc[q] = _tc_s2(z[q], u2p[q], y[q], k1[q], acc[q], dinvs[q], ws[q], b2s[q], tfs[q][1])
        for q in range(2):
            z[q] = sc_agg(u3p[q], *edges[q])
        for q in range(2):
            u4p[q], acc[q] = _tc_s3(z[q], u3p[q], y[q], m[q], acc[q], dinvs[q], ws[q], b2s[q], tfs[q][2])
        for q in range(2):
            z[q] = sc_agg(u4p[q], *edges[q])
        for q in range(2):
            y[q], yp[q] = _tc_s4(z[q], u4p[q], acc[q], dinvs[q], ws[q], b2s[q], tfs[q][3])
    return (y[0][:N], y[1][:N])


# speedup vs baseline: 1.0669x; 1.0192x over previous
import functools

import jax
import jax.numpy as jnp
from jax import lax
from jax.experimental import pallas as pl
from jax.experimental.pallas import tpu as pltpu
from jax.experimental.pallas import tpu_sc as plsc

N = 10000
D = 128
E = 160000

NC = 2
NS = 16
NW = NC * NS
N_PAD = 10240
E_PER_W = 5016
E_PAD = NW * E_PER_W
G = 152
NBATCH = E_PER_W // G
ROWS_PER_SUB = N_PAD // NS
DT = 0.1


def _sc_agg_body(uprime, src, dst, zout, sidx, didx, stag0, stag1, zsh,
                 isem, gsa, gsb, ssa, ssb):
    c = lax.axis_index("c")
    s = lax.axis_index("s")
    wid = c * NS + s
    ebase = wid * E_PER_W

    idx_descs = [
        pltpu.async_copy(src.at[pl.ds(ebase, E_PER_W)], sidx, isem),
        pltpu.async_copy(dst.at[pl.ds(ebase, E_PER_W)], didx, isem),
    ]

    def zero_row(i, _):
        for k in range(D // 16):
            stag0[i, pl.ds(k * 16, 16)] = jnp.zeros((16,), jnp.float32)
        return 0

    ZC = 128
    lax.fori_loop(0, ZC, zero_row, 0)
    zdescs = [
        pltpu.async_copy(stag0.at[pl.ds(0, ZC)],
                         zsh.at[pl.ds(s * ROWS_PER_SUB + j * ZC, ZC)], gsa)
        for j in range(ROWS_PER_SUB // ZC)
    ]
    for d in zdescs:
        d.wait()
    for d in idx_descs:
        d.wait()
    plsc.subcore_barrier()

    bufs = (stag0, stag1)
    gsem = (gsa, gsb)
    ssem = (ssa, ssb)
    gd = [None] * NBATCH
    sd = [None] * NBATCH
    gd[0] = pltpu.async_copy(uprime.at[sidx.at[pl.ds(0, G)]], bufs[0], gsem[0])
    for i in range(NBATCH):
        if i + 1 < NBATCH:
            if i >= 1:
                sd[i - 1].wait()
            b = (i + 1) % 2
            gd[i + 1] = pltpu.async_copy(
                uprime.at[sidx.at[pl.ds((i + 1) * G, G)]], bufs[b], gsem[b])
        gd[i].wait()
        b = i % 2
        sd[i] = pltpu.async_copy(
            bufs[b], zsh.at[didx.at[pl.ds(i * G, G)]], ssem[b], add=True)
    sd[NBATCH - 2].wait()
    sd[NBATCH - 1].wait()
    plsc.subcore_barrier()

    r0 = s * ROWS_PER_SUB
    pltpu.sync_copy(zsh.at[pl.ds(r0, ROWS_PER_SUB)], zout.at[c, pl.ds(r0, ROWS_PER_SUB)])


@functools.cache
def _get_sc_agg():
    return pl.kernel(
        _sc_agg_body,
        out_type=jax.ShapeDtypeStruct((NC, N_PAD, D), jnp.float32),
        mesh=plsc.VectorSubcoreMesh(
            core_axis_name="c", subcore_axis_name="s", num_cores=NC, num_subcores=NS
        ),
        scratch_types=[
            pltpu.VMEM((E_PER_W,), jnp.int32),
            pltpu.VMEM((E_PER_W,), jnp.int32),
            pltpu.VMEM((G, D), jnp.float32),
            pltpu.VMEM((G, D), jnp.float32),
            pltpu.VMEM_SHARED((N_PAD, D), jnp.float32),
            pltpu.SemaphoreType.DMA,
            pltpu.SemaphoreType.DMA,
            pltpu.SemaphoreType.DMA,
            pltpu.SemaphoreType.DMA,
            pltpu.SemaphoreType.DMA,
        ],
        name="gcn_edge_agg",
    )


BR = 640


def _psi(z_ref, up_ref, dinv_ref, w_ref, b_ref, tf_ref):
    g = dinv_ref[...] * (z_ref[0] + z_ref[1] + up_ref[...])
    h = jnp.dot(g, w_ref[...], preferred_element_type=jnp.float32) + b_ref[...]
    return jnp.maximum(h, 0.0) * tf_ref[...]


def _s1_body(z, yp, y, dinv, w, b, tf, k1_o, u2p_o, acc_o):
    k1 = _psi(z, yp, dinv, w, b, tf)
    k1_o[...] = k1
    u2p_o[...] = dinv[...] * (y[...] + (DT / 3.0) * k1)
    acc_o[...] = y[...] + (DT / 8.0) * k1


def _s2_body(z, u2p, y, k1, acc, dinv, w, b, tf, u3p_o, m_o, acc_o):
    k2 = _psi(z, u2p, dinv, w, b, tf)
    u3p_o[...] = dinv[...] * (y[...] + DT * k2 - (DT / 3.0) * k1[...])
    m_o[...] = k1[...] - k2
    acc_o[...] = acc[...] + (3.0 * DT / 8.0) * k2


def _s3_body(z, u3p, y, m, acc, dinv, w, b, tf, u4p_o, acc_o):
    k3 = _psi(z, u3p, dinv, w, b, tf)
    u4p_o[...] = dinv[...] * (y[...] + DT * (m[...] + k3))
    acc_o[...] = acc[...] + (3.0 * DT / 8.0) * k3


def _s4_body(z, u4p, acc, dinv, w, b, tf, y_o, yp_o):
    k4 = _psi(z, u4p, dinv, w, b, tf)
    y = acc[...] + (DT / 8.0) * k4
    y_o[...] = y
    yp_o[...] = dinv[...] * y


_row = pl.BlockSpec((BR, D), lambda i: (i, 0))
_zs = pl.BlockSpec((NC, BR, D), lambda i: (0, i, 0))
_dv = pl.BlockSpec((BR, 1), lambda i: (i, 0))
_wf = pl.BlockSpec((D, D), lambda i: (0, 0))
_vec = pl.BlockSpec((1, D), lambda i: (0, 0))
_grid = N_PAD // BR
_out = jax.ShapeDtypeStruct((N_PAD, D), jnp.float32)


def _stage_call(body, n_in, n_out):
    return pl.pallas_call(
        body,
        grid=(_grid,),
        in_specs=[_zs] + [_row] * n_in + [_dv, _wf, _vec, _vec],
        out_specs=[_row] * n_out,
        out_shape=[_out] * n_out,
    )


_tc_s1 = _stage_call(_s1_body, 2, 3)
_tc_s2 = _stage_call(_s2_body, 4, 3)
_tc_s3 = _stage_call(_s3_body, 4, 2)
_tc_s4 = _stage_call(_s4_body, 2, 2)


def _pad_edges(src, dst):
    pad_w = E_PER_W - E // NW
    pad_idx = (N + (jnp.arange(NW * pad_w, dtype=jnp.int32) % (N_PAD - N))
               ).reshape(NW, pad_w)
    src = jnp.concatenate([src.reshape(NW, E // NW), pad_idx], axis=1).reshape(-1)
    dst = jnp.concatenate([dst.reshape(NW, E // NW), pad_idx], axis=1).reshape(-1)
    return src, dst


def kernel(x, pos_edge_index, neg_edge_index, W_pos, b_pos, tw_pos, W_neg, b_neg, tw_neg):
    sc_agg = _get_sc_agg()
    x_pad = jnp.pad(x, ((0, N_PAD - N), (0, 0)))
    ones = jnp.ones((N_PAD, D), jnp.float32)

    edges, dinvs, b2s, tws, ws = [], [], [], [], []
    for src, dst, w, b, tw in (
        (pos_edge_index[0], pos_edge_index[1], W_pos, b_pos, tw_pos),
        (neg_edge_index[0], neg_edge_index[1], W_neg, b_neg, tw_neg),
    ):
        src, dst = _pad_edges(src, dst)
        zd = sc_agg(ones, src, dst)
        deg = zd[0, :, 0] + zd[1, :, 0] + 1.0
        edges.append((src, dst))
        dinvs.append(lax.rsqrt(deg)[:, None])
        b2s.append(b[None, :])
        tws.append(tw)
        ws.append(w)

    y = [x_pad, x_pad]
    yp = [dinvs[q] * x_pad for q in range(2)]
    k1 = [None, None]
    u2p = [None, None]
    u3p = [None, None]
    u4p = [None, None]
    m = [None, None]
    acc = [None, None]
    z = [None, None]
    for i in range(10):
        t0 = i * DT
        tfs = [[jax.nn.sigmoid((t0 + c * DT) * tws[q])[None, :]
                for c in (0.0, 1.0 / 3.0, 2.0 / 3.0, 1.0)] for q in range(2)]
        for q in range(2):
            z[q] = sc_agg(yp[q], *edges[q])
        for q in range(2):
            k1[q], u2p[q], acc[q] = _tc_s1(z[q], yp[q], y[q], dinvs[q], ws[q], b2s[q], tfs[q][0])
        for q in range(2):
            z[q] = sc_agg(u2p[q], *edges[q])
        for q in range(2):
            u3p[q], m[q], acc[q] = _tc_s2(z[q], u2p[q], y[q], k1[q], acc[q], dinvs[q], ws[q], b2s[q], tfs[q][1])
        for q in range(2):
            z[q] = sc_agg(u3p[q], *edges[q])
        for q in range(2):
            u4p[q], acc[q] = _tc_s3(z[q], u3p[q], y[q], m[q], acc[q], dinvs[q], ws[q], b2s[q], tfs[q][2])
        for q in range(2):
            z[q] = sc_agg(u4p[q], *edges[q])
        for q in range(2):
            y[q], yp[q] = _tc_s4(z[q], u4p[q], acc[q], dinvs[q], ws[q], b2s[q], tfs[q][3])
    return (y[0][:N], y[1][:N])


# speedup vs baseline: 1.0771x; 1.0095x over previous
import functools

import jax
import jax.numpy as jnp
from jax import lax
from jax.experimental import pallas as pl
from jax.experimental.pallas import tpu as pltpu
from jax.experimental.pallas import tpu_sc as plsc

N = 10000
D = 128
E = 160000

NC = 2
NS = 16
NW = NC * NS
N_PAD = 10240
E_PER_W = 5016
E_PAD = NW * E_PER_W
G = 152
NBATCH = E_PER_W // G
ROWS_PER_SUB = N_PAD // NS
DT = 0.1


def _sc_agg_body(uprime, src, dst, zout, sidx, didx, stag0, stag1, zsh,
                 isem, gsa, gsb, ssa, ssb):
    c = lax.axis_index("c")
    s = lax.axis_index("s")
    wid = c * NS + s
    ebase = wid * E_PER_W

    idx_descs = [
        pltpu.async_copy(src.at[pl.ds(ebase, E_PER_W)], sidx, isem),
        pltpu.async_copy(dst.at[pl.ds(ebase, E_PER_W)], didx, isem),
    ]

    for d in idx_descs:
        d.wait()

    bufs = (stag1, stag0)
    gsem = (gsb, gsa)
    ssem = (ssa, ssb)
    gd = [None] * NBATCH
    sd = [None] * NBATCH
    gd[0] = pltpu.async_copy(uprime.at[sidx.at[pl.ds(0, G)]], bufs[0], gsem[0])

    def zero_row(i, _):
        for k in range(D // 16):
            stag0[i, pl.ds(k * 16, 16)] = jnp.zeros((16,), jnp.float32)
        return 0

    ZC = 128
    lax.fori_loop(0, ZC, zero_row, 0)
    zdescs = [
        pltpu.async_copy(stag0.at[pl.ds(0, ZC)],
                         zsh.at[pl.ds(s * ROWS_PER_SUB + j * ZC, ZC)], gsa)
        for j in range(ROWS_PER_SUB // ZC)
    ]
    for d in zdescs:
        d.wait()
    plsc.subcore_barrier()

    for i in range(NBATCH):
        if i + 1 < NBATCH:
            if i >= 1:
                sd[i - 1].wait()
            b = (i + 1) % 2
            gd[i + 1] = pltpu.async_copy(
                uprime.at[sidx.at[pl.ds((i + 1) * G, G)]], bufs[b], gsem[b])
        gd[i].wait()
        b = i % 2
        sd[i] = pltpu.async_copy(
            bufs[b], zsh.at[didx.at[pl.ds(i * G, G)]], ssem[b], add=True)
    sd[NBATCH - 2].wait()
    sd[NBATCH - 1].wait()
    plsc.subcore_barrier()

    r0 = s * ROWS_PER_SUB
    pltpu.sync_copy(zsh.at[pl.ds(r0, ROWS_PER_SUB)], zout.at[c, pl.ds(r0, ROWS_PER_SUB)])


@functools.cache
def _get_sc_agg():
    return pl.kernel(
        _sc_agg_body,
        out_type=jax.ShapeDtypeStruct((NC, N_PAD, D), jnp.float32),
        mesh=plsc.VectorSubcoreMesh(
            core_axis_name="c", subcore_axis_name="s", num_cores=NC, num_subcores=NS
        ),
        scratch_types=[
            pltpu.VMEM((E_PER_W,), jnp.int32),
            pltpu.VMEM((E_PER_W,), jnp.int32),
            pltpu.VMEM((G, D), jnp.float32),
            pltpu.VMEM((G, D), jnp.float32),
            pltpu.VMEM_SHARED((N_PAD, D), jnp.float32),
            pltpu.SemaphoreType.DMA,
            pltpu.SemaphoreType.DMA,
            pltpu.SemaphoreType.DMA,
            pltpu.SemaphoreType.DMA,
            pltpu.SemaphoreType.DMA,
        ],
        name="gcn_edge_agg",
    )


BR = 640


def _psi(z_ref, up_ref, dinv_ref, w_ref, b_ref, tf_ref):
    g = dinv_ref[...] * (z_ref[0] + z_ref[1] + up_ref[...])
    h = jnp.dot(g, w_ref[...], preferred_element_type=jnp.float32) + b_ref[...]
    return jnp.maximum(h, 0.0) * tf_ref[...]


def _s1_body(z, yp, y, dinv, w, b, tf, k1_o, u2p_o, acc_o):
    k1 = _psi(z, yp, dinv, w, b, tf)
    k1_o[...] = k1
    u2p_o[...] = dinv[...] * (y[...] + (DT / 3.0) * k1)
    acc_o[...] = y[...] + (DT / 8.0) * k1


def _s2_body(z, u2p, y, k1, acc, dinv, w, b, tf, u3p_o, m_o, acc_o):
    k2 = _psi(z, u2p, dinv, w, b, tf)
    u3p_o[...] = dinv[...] * (y[...] + DT * k2 - (DT / 3.0) * k1[...])
    m_o[...] = k1[...] - k2
    acc_o[...] = acc[...] + (3.0 * DT / 8.0) * k2


def _s3_body(z, u3p, y, m, acc, dinv, w, b, tf, u4p_o, acc_o):
    k3 = _psi(z, u3p, dinv, w, b, tf)
    u4p_o[...] = dinv[...] * (y[...] + DT * (m[...] + k3))
    acc_o[...] = acc[...] + (3.0 * DT / 8.0) * k3


def _s4_body(z, u4p, acc, dinv, w, b, tf, y_o, yp_o):
    k4 = _psi(z, u4p, dinv, w, b, tf)
    y = acc[...] + (DT / 8.0) * k4
    y_o[...] = y
    yp_o[...] = dinv[...] * y


_row = pl.BlockSpec((BR, D), lambda i: (i, 0))
_zs = pl.BlockSpec((NC, BR, D), lambda i: (0, i, 0))
_dv = pl.BlockSpec((BR, 1), lambda i: (i, 0))
_wf = pl.BlockSpec((D, D), lambda i: (0, 0))
_vec = pl.BlockSpec((1, D), lambda i: (0, 0))
_grid = N_PAD // BR
_out = jax.ShapeDtypeStruct((N_PAD, D), jnp.float32)


def _stage_call(body, n_in, n_out):
    return pl.pallas_call(
        body,
        grid=(_grid,),
        in_specs=[_zs] + [_row] * n_in + [_dv, _wf, _vec, _vec],
        out_specs=[_row] * n_out,
        out_shape=[_out] * n_out,
    )


_tc_s1 = _stage_call(_s1_body, 2, 3)
_tc_s2 = _stage_call(_s2_body, 4, 3)
_tc_s3 = _stage_call(_s3_body, 4, 2)
_tc_s4 = _stage_call(_s4_body, 2, 2)


def _pad_edges(src, dst):
    pad_w = E_PER_W - E // NW
    pad_idx = (N + (jnp.arange(NW * pad_w, dtype=jnp.int32) % (N_PAD - N))
               ).reshape(NW, pad_w)
    src = jnp.concatenate([src.reshape(NW, E // NW), pad_idx], axis=1).reshape(-1)
    dst = jnp.concatenate([dst.reshape(NW, E // NW), pad_idx], axis=1).reshape(-1)
    return src, dst


def kernel(x, pos_edge_index, neg_edge_index, W_pos, b_pos, tw_pos, W_neg, b_neg, tw_neg):
    sc_agg = _get_sc_agg()
    x_pad = jnp.pad(x, ((0, N_PAD - N), (0, 0)))
    ones = jnp.ones((N_PAD, D), jnp.float32)

    edges, dinvs, b2s, tws, ws = [], [], [], [], []
    for src, dst, w, b, tw in (
        (pos_edge_index[0], pos_edge_index[1], W_pos, b_pos, tw_pos),
        (neg_edge_index[0], neg_edge_index[1], W_neg, b_neg, tw_neg),
    ):
        src, dst = _pad_edges(src, dst)
        zd = sc_agg(ones, src, dst)
        deg = zd[0, :, 0] + zd[1, :, 0] + 1.0
        edges.append((src, dst))
        dinvs.append(lax.rsqrt(deg)[:, None])
        b2s.append(b[None, :])
        tws.append(tw)
        ws.append(w)

    y = [x_pad, x_pad]
    yp = [dinvs[q] * x_pad for q in range(2)]
    k1 = [None, None]
    u2p = [None, None]
    u3p = [None, None]
    u4p = [None, None]
    m = [None, None]
    acc = [None, None]
    z = [None, None]
    for i in range(10):
        t0 = i * DT
        tfs = [[jax.nn.sigmoid((t0 + c * DT) * tws[q])[None, :]
                for c in (0.0, 1.0 / 3.0, 2.0 / 3.0, 1.0)] for q in range(2)]
        for q in range(2):
            z[q] = sc_agg(yp[q], *edges[q])
        for q in range(2):
            k1[q], u2p[q], acc[q] = _tc_s1(z[q], yp[q], y[q], dinvs[q], ws[q], b2s[q], tfs[q][0])
        for q in range(2):
            z[q] = sc_agg(u2p[q], *edges[q])
        for q in range(2):
            u3p[q], m[q], acc[q] = _tc_s2(z[q], u2p[q], y[q], k1[q], acc[q], dinvs[q], ws[q], b2s[q], tfs[q][1])
        for q in range(2):
            z[q] = sc_agg(u3p[q], *edges[q])
        for q in range(2):
            u4p[q], acc[q] = _tc_s3(z[q], u3p[q], y[q], m[q], acc[q], dinvs[q], ws[q], b2s[q], tfs[q][2])
        for q in range(2):
            z[q] = sc_agg(u4p[q], *edges[q])
        for q in range(2):
            y[q], yp[q] = _tc_s4(z[q], u4p[q], acc[q], dinvs[q], ws[q], b2s[q], tfs[q][3])
    return (y[0][:N], y[1][:N])
